# baseline (speedup 1.0000x reference)
import functools

import jax
import jax.numpy as jnp
from jax import lax
from jax.experimental import pallas as pl
from jax.experimental.pallas import tpu as pltpu
from jax.experimental.pallas import tpu_sc as plsc

N_NODES = 10000
N_EDGES = 320000
D = 128

NC = 2
NS = 16
NW = NC * NS
EDGES_PER_TILE = N_EDGES // NW
CHUNK = 80
NCHUNK = EDGES_PER_TILE // CHUNK
ZCH = 80
N_PAD = 10240
ROWS_PER_TILE = N_PAD // NS


def _sc_aggregate(x, src, dst, zacc, zcnt, ones):
    mesh = plsc.VectorSubcoreMesh(core_axis_name="c", subcore_axis_name="s")

    @functools.partial(
        pl.kernel,
        out_type=(
            jax.ShapeDtypeStruct((NC, N_PAD, D), jnp.float32),
            jax.ShapeDtypeStruct((NC, N_PAD), jnp.float32),
        ),
        mesh=mesh,
        scratch_types=[
            pltpu.VMEM_SHARED((N_PAD, D), jnp.float32),
            pltpu.VMEM_SHARED((N_PAD,), jnp.float32),
            pltpu.VMEM((ROWS_PER_TILE,), jnp.float32),
            [pltpu.VMEM((CHUNK,), jnp.int32) for _ in range(3)],
            [pltpu.VMEM((CHUNK,), jnp.int32) for _ in range(3)],
            [pltpu.VMEM((CHUNK, D), jnp.float32) for _ in range(3)],
            pltpu.VMEM((CHUNK,), jnp.float32),
            [pltpu.SemaphoreType.DMA for _ in range(3)],
            [pltpu.SemaphoreType.DMA for _ in range(3)],
            [pltpu.SemaphoreType.DMA for _ in range(3)],
        ],
    )
    def agg(x_hbm, src_hbm, dst_hbm, zacc_hbm, zcnt_hbm, ones_hbm,
            acc_out, cnt_out, acc_sh, cnt_sh, cntv,
            sidx, didx, rows, onesv, g, a, i):
        c = lax.axis_index("c")
        s = lax.axis_index("s")
        wid = c * NS + s
        row0 = s * ROWS_PER_TILE
        tile_base = wid * EDGES_PER_TILE

        def idx_start(k, sref, dref, sem):
            pltpu.async_copy(src_hbm.at[pl.ds(tile_base + k * CHUNK, CHUNK)], sref, sem)
            pltpu.async_copy(dst_hbm.at[pl.ds(tile_base + k * CHUNK, CHUNK)], dref, sem)

        def idx_wait(k, sref, dref, sem):
            pltpu.make_async_copy(src_hbm.at[pl.ds(tile_base + k * CHUNK, CHUNK)], sref, sem).wait()
            pltpu.make_async_copy(dst_hbm.at[pl.ds(tile_base + k * CHUNK, CHUNK)], dref, sem).wait()

        def scat_start(rref, dref, sem):
            pltpu.async_copy(rref, acc_sh.at[dref], sem, add=True)
            pltpu.async_copy(onesv, cnt_sh.at[dref], sem, add=True)

        def scat_drain(rref, dref, sem):
            pltpu.make_async_copy(rref, acc_sh.at[dref], sem).wait()
            pltpu.make_async_copy(onesv, cnt_sh.at[dref], sem).wait()

        pltpu.sync_copy(ones_hbm, onesv)
        pltpu.sync_copy(zcnt_hbm, cntv)
        pltpu.sync_copy(cntv, cnt_sh.at[pl.ds(row0, ROWS_PER_TILE)])
        for k in range(3):
            idx_start(k, sidx[k], didx[k], i[k])
        pltpu.sync_copy(zacc_hbm, rows[2].at[pl.ds(0, ZCH)])
        for k in range(ROWS_PER_TILE // ZCH):
            pltpu.async_copy(rows[2].at[pl.ds(0, ZCH)],
                             acc_sh.at[pl.ds(row0 + k * ZCH, ZCH)], a[0])
        for k in range(ROWS_PER_TILE // ZCH):
            pltpu.make_async_copy(rows[2].at[pl.ds(0, ZCH)],
                                  acc_sh.at[pl.ds(row0 + k * ZCH, ZCH)], a[0]).wait()

        plsc.subcore_barrier()

        for k in range(2):
            idx_wait(k, sidx[k], didx[k], i[k])
            pltpu.async_copy(x_hbm.at[sidx[k]], rows[k], g[k])

        @pl.loop(0, NCHUNK - 5, step=3)
        def _edges(j):
            pltpu.make_async_copy(x_hbm.at[sidx[0]], rows[0], g[0]).wait()
            scat_start(rows[0], didx[0], a[0])
            idx_wait(j + 2, sidx[2], didx[2], i[2])
            pltpu.async_copy(x_hbm.at[sidx[2]], rows[2], g[2])
            pltpu.make_async_copy(x_hbm.at[sidx[1]], rows[1], g[1]).wait()
            scat_start(rows[1], didx[1], a[1])
            scat_drain(rows[0], didx[0], a[0])
            idx_start(j + 3, sidx[0], didx[0], i[0])
            pltpu.make_async_copy(x_hbm.at[sidx[2]], rows[2], g[2]).wait()
            scat_start(rows[2], didx[2], a[2])
            scat_drain(rows[1], didx[1], a[1])
            idx_start(j + 4, sidx[1], didx[1], i[1])
            idx_wait(j + 3, sidx[0], didx[0], i[0])
            pltpu.async_copy(x_hbm.at[sidx[0]], rows[0], g[0])
            scat_drain(rows[2], didx[2], a[2])
            idx_start(j + 5, sidx[2], didx[2], i[2])
            idx_wait(j + 4, sidx[1], didx[1], i[1])
            pltpu.async_copy(x_hbm.at[sidx[1]], rows[1], g[1])

        pltpu.make_async_copy(x_hbm.at[sidx[0]], rows[0], g[0]).wait()
        scat_start(rows[0], didx[0], a[0])
        idx_wait(NCHUNK - 3, sidx[2], didx[2], i[2])
        pltpu.async_copy(x_hbm.at[sidx[2]], rows[2], g[2])
        pltpu.make_async_copy(x_hbm.at[sidx[1]], rows[1], g[1]).wait()
        scat_start(rows[1], didx[1], a[1])
        scat_drain(rows[0], didx[0], a[0])
        idx_start(NCHUNK - 2, sidx[0], didx[0], i[0])
        idx_wait(NCHUNK - 2, sidx[0], didx[0], i[0])
        pltpu.async_copy(x_hbm.at[sidx[0]], rows[0], g[0])
        pltpu.make_async_copy(x_hbm.at[sidx[2]], rows[2], g[2]).wait()
        scat_start(rows[2], didx[2], a[2])
        scat_drain(rows[1], didx[1], a[1])
        idx_start(NCHUNK - 1, sidx[1], didx[1], i[1])
        idx_wait(NCHUNK - 1, sidx[1], didx[1], i[1])
        pltpu.async_copy(x_hbm.at[sidx[1]], rows[1], g[1])
        pltpu.make_async_copy(x_hbm.at[sidx[0]], rows[0], g[0]).wait()
        scat_start(rows[0], didx[0], a[0])
        pltpu.make_async_copy(x_hbm.at[sidx[1]], rows[1], g[1]).wait()
        scat_start(rows[1], didx[1], a[1])
        scat_drain(rows[2], didx[2], a[2])
        scat_drain(rows[0], didx[0], a[0])
        scat_drain(rows[1], didx[1], a[1])

        plsc.subcore_barrier()
        pltpu.sync_copy(cnt_sh.at[pl.ds(row0, ROWS_PER_TILE)], cntv)
        pltpu.async_copy(cntv, cnt_out.at[c, pl.ds(row0, ROWS_PER_TILE)], a[1])
        pltpu.async_copy(acc_sh.at[pl.ds(row0, ROWS_PER_TILE)],
                         acc_out.at[c, pl.ds(row0, ROWS_PER_TILE)], a[0])
        pltpu.make_async_copy(acc_sh.at[pl.ds(row0, ROWS_PER_TILE)],
                              acc_out.at[c, pl.ds(row0, ROWS_PER_TILE)], a[0]).wait()
        pltpu.make_async_copy(cntv, cnt_out.at[c, pl.ds(row0, ROWS_PER_TILE)],
                              a[1]).wait()

    return agg(x, src, dst, zacc, zcnt, ones)


def _tc_finish(x, acc, cnt, W, b2):
    BLK = 1024
    nblk = N_PAD // BLK

    def body(x_ref, a0_ref, a1_ref, c_ref, w_ref, b_ref, o_ref):
        ssum = a0_ref[0] + a1_ref[0]
        n = c_ref[0] + c_ref[1]
        h = ssum / jnp.maximum(n, 1.0)[:, None]
        w0 = w_ref[0:D, :]
        w1 = w_ref[D:2 * D, :]
        o_ref[...] = (jnp.dot(x_ref[...], w0, preferred_element_type=jnp.float32)
                      + jnp.dot(h, w1, preferred_element_type=jnp.float32)
                      + b_ref[...])

    return pl.pallas_call(
        body,
        grid=(nblk,),
        in_specs=[
            pl.BlockSpec((BLK, D), lambda i: (i, 0)),
            pl.BlockSpec((1, BLK, D), lambda i: (0, i, 0)),
            pl.BlockSpec((1, BLK, D), lambda i: (1, i, 0)),
            pl.BlockSpec((NC, BLK), lambda i: (0, i)),
            pl.BlockSpec((2 * D, D), lambda i: (0, 0)),
            pl.BlockSpec((1, D), lambda i: (0, 0)),
        ],
        out_specs=pl.BlockSpec((BLK, D), lambda i: (i, 0)),
        out_shape=jax.ShapeDtypeStruct((N_NODES, D), jnp.float32),
    )(x, acc, acc, cnt, W, b2)


def kernel(x, edge_index, W, b):
    src = edge_index[0].astype(jnp.int32)
    dst = edge_index[1].astype(jnp.int32)
    zacc = jnp.zeros((ZCH, D), jnp.float32)
    zcnt = jnp.zeros((ROWS_PER_TILE,), jnp.float32)
    ones = jnp.ones((CHUNK,), jnp.float32)
    acc, cnt = _sc_aggregate(x, src, dst, zacc, zcnt, ones)
    return _tc_finish(x, acc, cnt, W, b.reshape(1, D))


# speedup vs baseline: 1.0102x; 1.0102x over previous
import functools

import jax
import jax.numpy as jnp
from jax import lax
from jax.experimental import pallas as pl
from jax.experimental.pallas import tpu as pltpu
from jax.experimental.pallas import tpu_sc as plsc

N_NODES = 10000
N_EDGES = 320000
D = 128

NC = 2
NS = 16
NW = NC * NS
EDGES_PER_TILE = N_EDGES // NW
CHUNK = 80
NCHUNK = EDGES_PER_TILE // CHUNK
ZCH = 80
N_PAD = 10240
ROWS_PER_TILE = N_PAD // NS


def _sc_aggregate(x, src, dst, zacc, zcnt, ones):
    mesh = plsc.VectorSubcoreMesh(core_axis_name="c", subcore_axis_name="s")

    @functools.partial(
        pl.kernel,
        out_type=(
            jax.ShapeDtypeStruct((NC, N_PAD, D), jnp.float32),
            jax.ShapeDtypeStruct((NC, N_PAD), jnp.float32),
        ),
        mesh=mesh,
        scratch_types=[
            pltpu.VMEM_SHARED((N_PAD, D), jnp.float32),
            pltpu.VMEM_SHARED((N_PAD,), jnp.float32),
            pltpu.VMEM((ROWS_PER_TILE,), jnp.float32),
            [pltpu.VMEM((CHUNK,), jnp.int32) for _ in range(3)],
            [pltpu.VMEM((CHUNK,), jnp.int32) for _ in range(3)],
            [pltpu.VMEM((CHUNK, D), jnp.float32) for _ in range(3)],
            pltpu.VMEM((CHUNK,), jnp.float32),
            [pltpu.SemaphoreType.DMA for _ in range(3)],
            [pltpu.SemaphoreType.DMA for _ in range(3)],
            [pltpu.SemaphoreType.DMA for _ in range(3)],
        ],
    )
    def agg(x_hbm, src_hbm, dst_hbm, zacc_hbm, zcnt_hbm, ones_hbm,
            acc_out, cnt_out, acc_sh, cnt_sh, cntv,
            sidx, didx, rows, onesv, g, a, i):
        c = lax.axis_index("c")
        s = lax.axis_index("s")
        wid = c * NS + s
        row0 = s * ROWS_PER_TILE
        tile_base = wid * EDGES_PER_TILE

        def idx_start(k, sref, dref, sem):
            pltpu.async_copy(src_hbm.at[pl.ds(tile_base + k * CHUNK, CHUNK)], sref, sem)
            pltpu.async_copy(dst_hbm.at[pl.ds(tile_base + k * CHUNK, CHUNK)], dref, sem)

        def idx_wait(k, sref, dref, sem):
            pltpu.make_async_copy(src_hbm.at[pl.ds(tile_base + k * CHUNK, CHUNK)], sref, sem).wait()
            pltpu.make_async_copy(dst_hbm.at[pl.ds(tile_base + k * CHUNK, CHUNK)], dref, sem).wait()

        def scat_start(rref, dref, sem):
            pltpu.async_copy(rref, acc_sh.at[dref], sem, add=True)
            pltpu.async_copy(onesv, cnt_sh.at[dref], sem, add=True)

        def scat_drain(rref, dref, sem):
            pltpu.make_async_copy(rref, acc_sh.at[dref], sem).wait()
            pltpu.make_async_copy(onesv, cnt_sh.at[dref], sem).wait()

        pltpu.sync_copy(ones_hbm, onesv)
        pltpu.sync_copy(zcnt_hbm, cntv)
        pltpu.sync_copy(cntv, cnt_sh.at[pl.ds(row0, ROWS_PER_TILE)])
        for k in range(3):
            idx_start(k, sidx[k], didx[k], i[k])
        pltpu.sync_copy(zacc_hbm, rows[2].at[pl.ds(0, ZCH)])
        for k in range(ROWS_PER_TILE // ZCH):
            pltpu.async_copy(rows[2].at[pl.ds(0, ZCH)],
                             acc_sh.at[pl.ds(row0 + k * ZCH, ZCH)], a[0])
        for k in range(ROWS_PER_TILE // ZCH):
            pltpu.make_async_copy(rows[2].at[pl.ds(0, ZCH)],
                                  acc_sh.at[pl.ds(row0 + k * ZCH, ZCH)], a[0]).wait()

        plsc.subcore_barrier()

        for k in range(2):
            idx_wait(k, sidx[k], didx[k], i[k])
            pltpu.async_copy(x_hbm.at[sidx[k]], rows[k], g[k])

        @pl.loop(0, NCHUNK - 5, step=3)
        def _edges(j):
            pltpu.make_async_copy(x_hbm.at[sidx[0]], rows[0], g[0]).wait()
            scat_start(rows[0], didx[0], a[0])
            idx_wait(j + 2, sidx[2], didx[2], i[2])
            pltpu.async_copy(x_hbm.at[sidx[2]], rows[2], g[2])
            pltpu.make_async_copy(x_hbm.at[sidx[1]], rows[1], g[1]).wait()
            scat_start(rows[1], didx[1], a[1])
            scat_drain(rows[0], didx[0], a[0])
            idx_start(j + 3, sidx[0], didx[0], i[0])
            pltpu.make_async_copy(x_hbm.at[sidx[2]], rows[2], g[2]).wait()
            scat_start(rows[2], didx[2], a[2])
            scat_drain(rows[1], didx[1], a[1])
            idx_start(j + 4, sidx[1], didx[1], i[1])
            idx_wait(j + 3, sidx[0], didx[0], i[0])
            pltpu.async_copy(x_hbm.at[sidx[0]], rows[0], g[0])
            scat_drain(rows[2], didx[2], a[2])
            idx_start(j + 5, sidx[2], didx[2], i[2])
            idx_wait(j + 4, sidx[1], didx[1], i[1])
            pltpu.async_copy(x_hbm.at[sidx[1]], rows[1], g[1])

        pltpu.make_async_copy(x_hbm.at[sidx[0]], rows[0], g[0]).wait()
        scat_start(rows[0], didx[0], a[0])
        idx_wait(NCHUNK - 3, sidx[2], didx[2], i[2])
        pltpu.async_copy(x_hbm.at[sidx[2]], rows[2], g[2])
        pltpu.make_async_copy(x_hbm.at[sidx[1]], rows[1], g[1]).wait()
        scat_start(rows[1], didx[1], a[1])
        scat_drain(rows[0], didx[0], a[0])
        idx_start(NCHUNK - 2, sidx[0], didx[0], i[0])
        idx_wait(NCHUNK - 2, sidx[0], didx[0], i[0])
        pltpu.async_copy(x_hbm.at[sidx[0]], rows[0], g[0])
        pltpu.make_async_copy(x_hbm.at[sidx[2]], rows[2], g[2]).wait()
        scat_start(rows[2], didx[2], a[2])
        scat_drain(rows[1], didx[1], a[1])
        idx_start(NCHUNK - 1, sidx[1], didx[1], i[1])
        idx_wait(NCHUNK - 1, sidx[1], didx[1], i[1])
        pltpu.async_copy(x_hbm.at[sidx[1]], rows[1], g[1])
        pltpu.make_async_copy(x_hbm.at[sidx[0]], rows[0], g[0]).wait()
        scat_start(rows[0], didx[0], a[0])
        pltpu.make_async_copy(x_hbm.at[sidx[1]], rows[1], g[1]).wait()
        scat_start(rows[1], didx[1], a[1])
        scat_drain(rows[2], didx[2], a[2])
        scat_drain(rows[0], didx[0], a[0])
        scat_drain(rows[1], didx[1], a[1])

        plsc.subcore_barrier()
        pltpu.sync_copy(cnt_sh.at[pl.ds(row0, ROWS_PER_TILE)], cntv)
        pltpu.async_copy(cntv, cnt_out.at[c, pl.ds(row0, ROWS_PER_TILE)], a[1])
        nwb = ROWS_PER_TILE // ZCH

        def wb_src(k):
            return acc_sh.at[pl.ds(row0 + k * ZCH, ZCH)]

        def wb_dst(k):
            return acc_out.at[c, pl.ds(row0 + k * ZCH, ZCH)]

        for k in range(nwb):
            r = rows[k % 3].at[pl.ds(0, ZCH)]
            if k >= 3:
                pltpu.make_async_copy(rows[k % 3].at[pl.ds(0, ZCH)],
                                      wb_dst(k - 3), a[0]).wait()
            pltpu.sync_copy(wb_src(k), r)
            pltpu.async_copy(r, wb_dst(k), a[0])
        for k in range(nwb - 3, nwb):
            pltpu.make_async_copy(rows[k % 3].at[pl.ds(0, ZCH)],
                                  wb_dst(k), a[0]).wait()
        pltpu.make_async_copy(cntv, cnt_out.at[c, pl.ds(row0, ROWS_PER_TILE)],
                              a[1]).wait()

    return agg(x, src, dst, zacc, zcnt, ones)


def _tc_finish(x, acc, cnt, W, b2):
    BLK = 1024
    nblk = N_PAD // BLK

    def body(x_ref, a0_ref, a1_ref, c_ref, w_ref, b_ref, o_ref):
        ssum = a0_ref[0] + a1_ref[0]
        n = c_ref[0] + c_ref[1]
        h = ssum / jnp.maximum(n, 1.0)[:, None]
        w0 = w_ref[0:D, :]
        w1 = w_ref[D:2 * D, :]
        o_ref[...] = (jnp.dot(x_ref[...], w0, preferred_element_type=jnp.float32)
                      + jnp.dot(h, w1, preferred_element_type=jnp.float32)
                      + b_ref[...])

    return pl.pallas_call(
        body,
        grid=(nblk,),
        in_specs=[
            pl.BlockSpec((BLK, D), lambda i: (i, 0)),
            pl.BlockSpec((1, BLK, D), lambda i: (0, i, 0)),
            pl.BlockSpec((1, BLK, D), lambda i: (1, i, 0)),
            pl.BlockSpec((NC, BLK), lambda i: (0, i)),
            pl.BlockSpec((2 * D, D), lambda i: (0, 0)),
            pl.BlockSpec((1, D), lambda i: (0, 0)),
        ],
        out_specs=pl.BlockSpec((BLK, D), lambda i: (i, 0)),
        out_shape=jax.ShapeDtypeStruct((N_NODES, D), jnp.float32),
    )(x, acc, acc, cnt, W, b2)


def kernel(x, edge_index, W, b):
    src = edge_index[0].astype(jnp.int32)
    dst = edge_index[1].astype(jnp.int32)
    zacc = jnp.zeros((ZCH, D), jnp.float32)
    zcnt = jnp.zeros((ROWS_PER_TILE,), jnp.float32)
    ones = jnp.ones((CHUNK,), jnp.float32)
    acc, cnt = _sc_aggregate(x, src, dst, zacc, zcnt, ones)
    return _tc_finish(x, acc, cnt, W, b.reshape(1, D))
